# baseline (speedup 1.0000x reference)
import functools

import numpy as np
import jax
import jax.numpy as jnp
from jax.experimental import pallas as pl
from jax.experimental.pallas import tpu as pltpu
from jax.experimental.pallas import tpu_sc as plsc

N = 2048
D = 768
H = 3072
E = 8
NSLOT = 2
A = N * NSLOT
TM = 128
NT = A // TM + E
PAD = NT * TM
NH = 4
HC = H // NH
CH = 128
NCH = A // CH
SCW = 128
RSPLIT = 3
PACKW = D // 2 // RSPLIT
DSUB = D // RSPLIT
NEG_INF = -1e30


def _gate_route_kernel(xf_ref, wgt_ref, t128_ref, w_full_ref, w1c_ref,
                       w2c_ref, posa_ref, eot_ref, valid_ref):
    logits = jax.lax.dot_general(
        xf_ref[...], wgt_ref[...], (((1,), (0,)), ((), ())),
        preferred_element_type=jnp.float32)
    col = jax.lax.broadcasted_iota(jnp.int32, (N, E), 1)
    m1 = jnp.max(logits, axis=1, keepdims=True)
    i1 = jnp.min(jnp.where(logits == m1, col, E), axis=1, keepdims=True)
    masked = jnp.where(col == i1, NEG_INF, logits)
    m2 = jnp.max(masked, axis=1, keepdims=True)
    i2 = jnp.min(jnp.where(masked == m2, col, E), axis=1, keepdims=True)
    e2 = jnp.exp(m2 - m1)
    denom = 1.0 + e2
    w1 = 1.0 / denom
    w2 = e2 / denom
    oh1 = (col == i1).astype(jnp.float32)
    oh2 = (col == i2).astype(jnp.float32)
    w_full_ref[...] = w1 * oh1 + w2 * oh2
    w1c_ref[...] = w1
    w2c_ref[...] = w2

    t128 = t128_ref[...]
    base = jnp.zeros((1, E), jnp.float32)
    ranks = []
    blks = []
    for c in range(NCH):
        src = oh1 if c < NCH // 2 else oh2
        r0 = (c % (NCH // 2)) * CH
        blk = jax.lax.slice(src, (r0, 0), (r0 + CH, E))
        blks.append(blk)
        r = jax.lax.dot_general(
            t128, blk.astype(jnp.bfloat16), (((1,), (0,)), ((), ())),
            preferred_element_type=jnp.float32) + base
        ranks.append(jnp.sum(r * blk, axis=1, keepdims=True))
        base = base + jnp.sum(blk, axis=0, keepdims=True)
    counts = base
    tiles_e = jnp.floor((counts + (TM - 1)) * (1.0 / TM))
    ecol = jax.lax.broadcasted_iota(jnp.int32, (E, E), 1)
    erow = jax.lax.broadcasted_iota(jnp.int32, (E, E), 0)
    tril_e = (erow < ecol).astype(jnp.float32)
    toff = jax.lax.dot_general(
        tiles_e, tril_e, (((1,), (0,)), ((), ())),
        preferred_element_type=jnp.float32)
    off_pad = toff * float(TM)
    for c in range(NCH):
        pos = jnp.sum(blks[c] * off_pad, axis=1, keepdims=True) + ranks[c]
        posa_ref[c] = pos.astype(jnp.int32)
    trow = jax.lax.broadcasted_iota(jnp.int32, (NT, E), 0).astype(jnp.float32)
    ge = (trow >= toff).astype(jnp.float32)
    eot = jnp.sum(ge, axis=1, keepdims=True) - 1.0
    total_tiles = jnp.sum(tiles_e, axis=1, keepdims=True)
    tcol = jax.lax.broadcasted_iota(jnp.int32, (NT, 1), 0).astype(jnp.float32)
    eot_ref[...] = jnp.clip(eot, 0.0, E - 1).astype(jnp.int32)
    valid_ref[...] = (tcol < total_tiles).astype(jnp.int32)


def _ffn1_kernel(eot_ref, valid_ref, xs_ref, w1_ref, b1_ref, h_ref, w1s_ref):
    t = pl.program_id(0)
    prev = eot_ref[jnp.maximum(t - 1, 0)]
    changed = jnp.logical_or(t == 0, eot_ref[t] != prev)

    @pl.when(changed)
    def _():
        w1s_ref[...] = w1_ref[0].astype(jnp.bfloat16)

    @pl.when(valid_ref[t] == 1)
    def _():
        h = jax.lax.dot_general(
            xs_ref[...].astype(jnp.bfloat16), w1s_ref[...],
            (((1,), (1,)), ((), ())),
            preferred_element_type=jnp.float32)
        h = h + b1_ref[0]
        h_ref[...] = (0.5 * h * (1.0 + jax.lax.erf(h * 0.7071067811865476))
                      ).astype(jnp.bfloat16)


def _ffn2_kernel(eot_ref, valid_ref, h_ref, w2_ref, b2_ref, ys_ref, w2s_ref):
    t = pl.program_id(0)
    prev = eot_ref[jnp.maximum(t - 1, 0)]
    changed = jnp.logical_or(t == 0, eot_ref[t] != prev)

    @pl.when(changed)
    def _():
        w2s_ref[...] = w2_ref[0].astype(jnp.bfloat16)

    @pl.when(valid_ref[t] == 1)
    def _():
        eo = jax.lax.dot_general(
            h_ref[...], w2s_ref[...], (((1,), (1,)), ((), ())),
            preferred_element_type=jnp.float32)
        ys_ref[...] = eo + b2_ref[0]


def _combine_kernel(y01_ref, w1c_ref, w2c_ref, out_ref):
    out_ref[...] = (w1c_ref[...] * y01_ref[:N] +
                    w2c_ref[...] * y01_ref[N:])


_vector_mesh = functools.partial(
    plsc.VectorSubcoreMesh, core_axis_name="c", subcore_axis_name="s")


def _sc_dispatch(xf, dest3):
    xv = xf.reshape(N * RSPLIT, DSUB)
    nwin = N * RSPLIT // SCW
    @pl.kernel(out_type=jax.ShapeDtypeStruct((PAD * RSPLIT, DSUB),
                                             jnp.float32),
               mesh=_vector_mesh())
    def disp(xv_hbm, d3_hbm, xs_hbm):
        def body(x_vmem, i_vmem):
            pltpu.sync_copy(x_vmem, xs_hbm.at[i_vmem.at[0]])

        pltpu.emit_pipeline(
            body,
            grid=(2 * nwin,),
            in_specs=[
                pl.BlockSpec((SCW, DSUB), lambda i: (i % nwin, 0)),
                pl.BlockSpec((1, SCW), lambda i: (0, i)),
            ],
            out_specs=[],
            core_axis_name=("c", "s"),
            dimension_semantics=(pltpu.PARALLEL,),
        )(xv_hbm, d3_hbm)

    return disp(xv, dest3).reshape(PAD, D)


def _sc_combine(ys, src3):
    yv = ys.reshape(PAD * RSPLIT, DSUB)
    nwin = NSLOT * N * RSPLIT // SCW
    @pl.kernel(out_type=jax.ShapeDtypeStruct((NSLOT * N * RSPLIT, DSUB),
                                             jnp.float32),
               mesh=_vector_mesh())
    def comb(yv_hbm, s3_hbm, y_hbm):
        def body(i_vmem, y_vmem):
            pltpu.sync_copy(yv_hbm.at[i_vmem.at[0]], y_vmem)

        pltpu.emit_pipeline(
            body,
            grid=(nwin,),
            in_specs=[
                pl.BlockSpec((1, SCW), lambda i: (0, i)),
            ],
            out_specs=[
                pl.BlockSpec((SCW, DSUB), lambda i: (i, 0)),
            ],
            core_axis_name=("c", "s"),
            dimension_semantics=(pltpu.PARALLEL,),
        )(s3_hbm, y_hbm)

    return comb(yv, src3).reshape(NSLOT * N, D)


_T128 = np.tril(np.ones((CH, CH), np.float32), -1)


def kernel(x, Wg, W1, b1, W2, b2):
    b, s, d = x.shape
    xf = x.reshape(N, D)

    t128 = jnp.asarray(_T128, jnp.bfloat16)
    w_full, w1c, w2c, posa, eot, valid = pl.pallas_call(
        _gate_route_kernel,
        out_shape=(
            jax.ShapeDtypeStruct((N, E), jnp.float32),
            jax.ShapeDtypeStruct((N, 1), jnp.float32),
            jax.ShapeDtypeStruct((N, 1), jnp.float32),
            jax.ShapeDtypeStruct((NCH, CH, 1), jnp.int32),
            jax.ShapeDtypeStruct((NT, 1), jnp.int32),
            jax.ShapeDtypeStruct((NT, 1), jnp.int32),
        ),
    )(xf, Wg.T, t128)

    posf = posa.reshape(A)
    r3 = jnp.arange(RSPLIT, dtype=jnp.int32)
    idx3 = (posf[:, None] * RSPLIT + r3[None, :]).reshape(1, A * RSPLIT)

    xs = _sc_dispatch(xf, idx3)

    b1r = b1.reshape(E, 1, H)
    b2r = b2.reshape(E, 1, D)
    eotf = eot.reshape(NT)
    validf = valid.reshape(NT)
    hs = pl.pallas_call(
        _ffn1_kernel,
        grid_spec=pltpu.PrefetchScalarGridSpec(
            num_scalar_prefetch=2,
            grid=(NT,),
            in_specs=[
                pl.BlockSpec((TM, D), lambda t, eot, valid: (t, 0)),
                pl.BlockSpec((1, H, D), lambda t, eot, valid: (eot[t], 0, 0)),
                pl.BlockSpec((1, 1, H), lambda t, eot, valid: (eot[t], 0, 0)),
            ],
            out_specs=pl.BlockSpec((TM, H), lambda t, eot, valid: (t, 0)),
            scratch_shapes=[pltpu.VMEM((H, D), jnp.bfloat16)],
        ),
        out_shape=jax.ShapeDtypeStruct((PAD, H), jnp.bfloat16),
    )(eotf, validf, xs, W1, b1r)

    ys = pl.pallas_call(
        _ffn2_kernel,
        grid_spec=pltpu.PrefetchScalarGridSpec(
            num_scalar_prefetch=2,
            grid=(NT,),
            in_specs=[
                pl.BlockSpec((TM, H), lambda t, eot, valid: (t, 0)),
                pl.BlockSpec((1, D, H), lambda t, eot, valid: (eot[t], 0, 0)),
                pl.BlockSpec((1, 1, D), lambda t, eot, valid: (eot[t], 0, 0)),
            ],
            out_specs=pl.BlockSpec((TM, D), lambda t, eot, valid: (t, 0)),
            scratch_shapes=[pltpu.VMEM((D, H), jnp.bfloat16)],
        ),
        out_shape=jax.ShapeDtypeStruct((PAD, D), jnp.float32),
    )(eotf, validf, hs, W2, b2r)

    y01 = _sc_combine(ys, idx3)

    out = pl.pallas_call(
        _combine_kernel,
        out_shape=jax.ShapeDtypeStruct((N, D), jnp.float32),
    )(y01, w1c, w2c)

    return out.reshape(b, s, d), w_full


# speedup vs baseline: 1.1930x; 1.1930x over previous
import functools

import numpy as np
import jax
import jax.numpy as jnp
from jax.experimental import pallas as pl
from jax.experimental.pallas import tpu as pltpu
from jax.experimental.pallas import tpu_sc as plsc

N = 2048
D = 768
H = 3072
E = 8
NSLOT = 2
A = N * NSLOT
TM = 256
NT = A // TM + E
PAD = NT * TM
CH = 128
NCH = A // CH
SCW = 128
RSPLIT = 3
DSUB = D // RSPLIT
NEG_INF = -1e30


def _gate_route_kernel(xf_ref, wgt_ref, t128_ref, w_full_ref, w1c_ref,
                       w2c_ref, posa_ref, eot_ref, valid_ref):
    logits = jax.lax.dot_general(
        xf_ref[...], wgt_ref[...], (((1,), (0,)), ((), ())),
        preferred_element_type=jnp.float32)
    col = jax.lax.broadcasted_iota(jnp.int32, (N, E), 1)
    m1 = jnp.max(logits, axis=1, keepdims=True)
    i1 = jnp.min(jnp.where(logits == m1, col, E), axis=1, keepdims=True)
    masked = jnp.where(col == i1, NEG_INF, logits)
    m2 = jnp.max(masked, axis=1, keepdims=True)
    i2 = jnp.min(jnp.where(masked == m2, col, E), axis=1, keepdims=True)
    e2 = jnp.exp(m2 - m1)
    denom = 1.0 + e2
    w1 = 1.0 / denom
    w2 = e2 / denom
    oh1 = (col == i1).astype(jnp.float32)
    oh2 = (col == i2).astype(jnp.float32)
    w_full_ref[...] = w1 * oh1 + w2 * oh2
    w1c_ref[...] = w1
    w2c_ref[...] = w2

    t128 = t128_ref[...]
    base = jnp.zeros((1, E), jnp.float32)
    ranks = []
    blks = []
    for c in range(NCH):
        src = oh1 if c < NCH // 2 else oh2
        r0 = (c % (NCH // 2)) * CH
        blk = jax.lax.slice(src, (r0, 0), (r0 + CH, E))
        blks.append(blk)
        r = jax.lax.dot_general(
            t128, blk.astype(jnp.bfloat16), (((1,), (0,)), ((), ())),
            preferred_element_type=jnp.float32) + base
        ranks.append(jnp.sum(r * blk, axis=1, keepdims=True))
        base = base + jnp.sum(blk, axis=0, keepdims=True)
    counts = base
    tiles_e = jnp.floor((counts + (TM - 1)) * (1.0 / TM))
    ecol = jax.lax.broadcasted_iota(jnp.int32, (E, E), 1)
    erow = jax.lax.broadcasted_iota(jnp.int32, (E, E), 0)
    tril_e = (erow < ecol).astype(jnp.float32)
    toff = jax.lax.dot_general(
        tiles_e, tril_e, (((1,), (0,)), ((), ())),
        preferred_element_type=jnp.float32)
    off_pad = toff * float(TM)
    for c in range(NCH):
        pos = jnp.sum(blks[c] * off_pad, axis=1, keepdims=True) + ranks[c]
        posa_ref[c] = pos.astype(jnp.int32)
    trow = jax.lax.broadcasted_iota(jnp.int32, (NT, E), 0).astype(jnp.float32)
    ge = (trow >= toff).astype(jnp.float32)
    eot = jnp.sum(ge, axis=1, keepdims=True) - 1.0
    total_tiles = jnp.sum(tiles_e, axis=1, keepdims=True)
    tcol = jax.lax.broadcasted_iota(jnp.int32, (NT, 1), 0).astype(jnp.float32)
    eot_ref[...] = jnp.clip(eot, 0.0, E - 1).astype(jnp.int32)
    valid_ref[...] = (tcol < total_tiles).astype(jnp.int32)


def _ffn1_kernel(eot_ref, valid_ref, xs_ref, w1_ref, b1_ref, h_ref, w1s_ref):
    t = pl.program_id(0)
    prev = eot_ref[jnp.maximum(t - 1, 0)]
    changed = jnp.logical_or(t == 0, eot_ref[t] != prev)

    @pl.when(changed)
    def _():
        w1s_ref[...] = w1_ref[0].astype(jnp.bfloat16)

    @pl.when(valid_ref[t] == 1)
    def _():
        h = jax.lax.dot_general(
            xs_ref[...].astype(jnp.bfloat16), w1s_ref[...],
            (((1,), (1,)), ((), ())),
            preferred_element_type=jnp.float32)
        h = h + b1_ref[0]
        h_ref[...] = (0.5 * h * (1.0 + jax.lax.erf(h * 0.7071067811865476))
                      ).astype(jnp.bfloat16)


def _ffn2_kernel(eot_ref, valid_ref, h_ref, w2_ref, b2_ref, ys_ref, w2s_ref):
    t = pl.program_id(0)
    prev = eot_ref[jnp.maximum(t - 1, 0)]
    changed = jnp.logical_or(t == 0, eot_ref[t] != prev)

    @pl.when(changed)
    def _():
        w2s_ref[...] = w2_ref[0].astype(jnp.bfloat16)

    @pl.when(valid_ref[t] == 1)
    def _():
        eo = jax.lax.dot_general(
            h_ref[...], w2s_ref[...], (((1,), (1,)), ((), ())),
            preferred_element_type=jnp.float32)
        ys_ref[...] = eo + b2_ref[0]


def _combine_kernel(y01_ref, w1c_ref, w2c_ref, out_ref):
    out_ref[...] = (w1c_ref[...] * y01_ref[:N] +
                    w2c_ref[...] * y01_ref[N:])


_vector_mesh = functools.partial(
    plsc.VectorSubcoreMesh, core_axis_name="c", subcore_axis_name="s")


def _sc_dispatch(xf, dest3):
    xv = xf.reshape(N * RSPLIT, DSUB)
    nwin = N * RSPLIT // SCW
    @pl.kernel(out_type=jax.ShapeDtypeStruct((PAD * RSPLIT, DSUB),
                                             jnp.float32),
               mesh=_vector_mesh())
    def disp(xv_hbm, d3_hbm, xs_hbm):
        def body(x_vmem, i_vmem):
            pltpu.sync_copy(x_vmem, xs_hbm.at[i_vmem.at[0]])

        pltpu.emit_pipeline(
            body,
            grid=(2 * nwin,),
            in_specs=[
                pl.BlockSpec((SCW, DSUB), lambda i: (i % nwin, 0)),
                pl.BlockSpec((1, SCW), lambda i: (0, i)),
            ],
            out_specs=[],
            core_axis_name=("c", "s"),
            dimension_semantics=(pltpu.PARALLEL,),
        )(xv_hbm, d3_hbm)

    return disp(xv, dest3).reshape(PAD, D)


def _sc_combine(ys, src3):
    yv = ys.reshape(PAD * RSPLIT, DSUB)
    nwin = NSLOT * N * RSPLIT // SCW
    @pl.kernel(out_type=jax.ShapeDtypeStruct((NSLOT * N * RSPLIT, DSUB),
                                             jnp.float32),
               mesh=_vector_mesh())
    def comb(yv_hbm, s3_hbm, y_hbm):
        def body(i_vmem, y_vmem):
            pltpu.sync_copy(yv_hbm.at[i_vmem.at[0]], y_vmem)

        pltpu.emit_pipeline(
            body,
            grid=(nwin,),
            in_specs=[
                pl.BlockSpec((1, SCW), lambda i: (0, i)),
            ],
            out_specs=[
                pl.BlockSpec((SCW, DSUB), lambda i: (i, 0)),
            ],
            core_axis_name=("c", "s"),
            dimension_semantics=(pltpu.PARALLEL,),
        )(s3_hbm, y_hbm)

    return comb(yv, src3).reshape(NSLOT * N, D)


_T128 = np.tril(np.ones((CH, CH), np.float32), -1)


def kernel(x, Wg, W1, b1, W2, b2):
    b, s, d = x.shape
    xf = x.reshape(N, D)

    t128 = jnp.asarray(_T128, jnp.bfloat16)
    w_full, w1c, w2c, posa, eot, valid = pl.pallas_call(
        _gate_route_kernel,
        out_shape=(
            jax.ShapeDtypeStruct((N, E), jnp.float32),
            jax.ShapeDtypeStruct((N, 1), jnp.float32),
            jax.ShapeDtypeStruct((N, 1), jnp.float32),
            jax.ShapeDtypeStruct((NCH, CH, 1), jnp.int32),
            jax.ShapeDtypeStruct((NT, 1), jnp.int32),
            jax.ShapeDtypeStruct((NT, 1), jnp.int32),
        ),
    )(xf, Wg.T, t128)

    posf = posa.reshape(A)
    r3 = jnp.arange(RSPLIT, dtype=jnp.int32)
    idx3 = (posf[:, None] * RSPLIT + r3[None, :]).reshape(1, A * RSPLIT)

    xs = _sc_dispatch(xf, idx3)

    b1r = b1.reshape(E, 1, H)
    b2r = b2.reshape(E, 1, D)
    eotf = eot.reshape(NT)
    validf = valid.reshape(NT)
    hs = pl.pallas_call(
        _ffn1_kernel,
        grid_spec=pltpu.PrefetchScalarGridSpec(
            num_scalar_prefetch=2,
            grid=(NT,),
            in_specs=[
                pl.BlockSpec((TM, D), lambda t, eot, valid: (t, 0)),
                pl.BlockSpec((1, H, D), lambda t, eot, valid: (eot[t], 0, 0)),
                pl.BlockSpec((1, 1, H), lambda t, eot, valid: (eot[t], 0, 0)),
            ],
            out_specs=pl.BlockSpec((TM, H), lambda t, eot, valid: (t, 0)),
            scratch_shapes=[pltpu.VMEM((H, D), jnp.bfloat16)],
        ),
        out_shape=jax.ShapeDtypeStruct((PAD, H), jnp.bfloat16),
    )(eotf, validf, xs, W1, b1r)

    ys = pl.pallas_call(
        _ffn2_kernel,
        grid_spec=pltpu.PrefetchScalarGridSpec(
            num_scalar_prefetch=2,
            grid=(NT,),
            in_specs=[
                pl.BlockSpec((TM, H), lambda t, eot, valid: (t, 0)),
                pl.BlockSpec((1, D, H), lambda t, eot, valid: (eot[t], 0, 0)),
                pl.BlockSpec((1, 1, D), lambda t, eot, valid: (eot[t], 0, 0)),
            ],
            out_specs=pl.BlockSpec((TM, D), lambda t, eot, valid: (t, 0)),
            scratch_shapes=[pltpu.VMEM((D, H), jnp.bfloat16)],
        ),
        out_shape=jax.ShapeDtypeStruct((PAD, D), jnp.float32),
    )(eotf, validf, hs, W2, b2r)

    y01 = _sc_combine(ys, idx3)

    out = pl.pallas_call(
        _combine_kernel,
        out_shape=jax.ShapeDtypeStruct((N, D), jnp.float32),
    )(y01, w1c, w2c)

    return out.reshape(b, s, d), w_full


# speedup vs baseline: 1.1982x; 1.0043x over previous
import functools

import numpy as np
import jax
import jax.numpy as jnp
from jax.experimental import pallas as pl
from jax.experimental.pallas import tpu as pltpu
from jax.experimental.pallas import tpu_sc as plsc

N = 2048
D = 768
H = 3072
E = 8
NSLOT = 2
A = N * NSLOT
TM = 256
NT = A // TM + E
PAD = NT * TM
CH = 128
NCH = A // CH
SCW = 128
RSPLIT = 3
DSUB = D // RSPLIT
NEG_INF = -1e30


def _gate_route_kernel(xf_ref, wgt_ref, t128_ref, w_full_ref, w1c_ref,
                       w2c_ref, posa_ref, eot_ref, valid_ref):
    logits = jax.lax.dot_general(
        xf_ref[...], wgt_ref[...], (((1,), (0,)), ((), ())),
        preferred_element_type=jnp.float32)
    col = jax.lax.broadcasted_iota(jnp.int32, (N, E), 1)
    m1 = jnp.max(logits, axis=1, keepdims=True)
    i1 = jnp.min(jnp.where(logits == m1, col, E), axis=1, keepdims=True)
    masked = jnp.where(col == i1, NEG_INF, logits)
    m2 = jnp.max(masked, axis=1, keepdims=True)
    i2 = jnp.min(jnp.where(masked == m2, col, E), axis=1, keepdims=True)
    e2 = jnp.exp(m2 - m1)
    denom = 1.0 + e2
    w1 = 1.0 / denom
    w2 = e2 / denom
    oh1 = (col == i1).astype(jnp.float32)
    oh2 = (col == i2).astype(jnp.float32)
    w_full_ref[...] = w1 * oh1 + w2 * oh2
    w1c_ref[...] = w1
    w2c_ref[...] = w2

    t128 = t128_ref[...]
    base = jnp.zeros((1, E), jnp.float32)
    ranks = []
    blks = []
    for c in range(NCH):
        src = oh1 if c < NCH // 2 else oh2
        r0 = (c % (NCH // 2)) * CH
        blk = jax.lax.slice(src, (r0, 0), (r0 + CH, E))
        blks.append(blk)
        r = jax.lax.dot_general(
            t128, blk.astype(jnp.bfloat16), (((1,), (0,)), ((), ())),
            preferred_element_type=jnp.float32) + base
        ranks.append(jnp.sum(r * blk, axis=1, keepdims=True))
        base = base + jnp.sum(blk, axis=0, keepdims=True)
    counts = base
    tiles_e = jnp.floor((counts + (TM - 1)) * (1.0 / TM))
    ecol = jax.lax.broadcasted_iota(jnp.int32, (E, E), 1)
    erow = jax.lax.broadcasted_iota(jnp.int32, (E, E), 0)
    tril_e = (erow < ecol).astype(jnp.float32)
    toff = jax.lax.dot_general(
        tiles_e, tril_e, (((1,), (0,)), ((), ())),
        preferred_element_type=jnp.float32)
    off_pad = toff * float(TM)
    iota3 = jax.lax.broadcasted_iota(jnp.int32, (CH, RSPLIT), 1)
    iota3f = iota3.astype(jnp.float32)
    for c in range(NCH):
        pos = jnp.sum(blks[c] * off_pad, axis=1, keepdims=True) + ranks[c]
        posa_ref[c] = (pos * float(RSPLIT) + iota3f).astype(jnp.int32)
    trow = jax.lax.broadcasted_iota(jnp.int32, (NT, E), 0).astype(jnp.float32)
    ge = (trow >= toff).astype(jnp.float32)
    eot = jnp.sum(ge, axis=1, keepdims=True) - 1.0
    total_tiles = jnp.sum(tiles_e, axis=1, keepdims=True)
    tcol = jax.lax.broadcasted_iota(jnp.int32, (NT, 1), 0).astype(jnp.float32)
    eot_ref[...] = jnp.clip(eot, 0.0, E - 1).astype(jnp.int32)
    valid_ref[...] = (tcol < total_tiles).astype(jnp.int32)


def _ffn1_kernel(eot_ref, valid_ref, xs_ref, w1_ref, b1_ref, h_ref, w1s_ref):
    t = pl.program_id(0)
    prev = eot_ref[jnp.maximum(t - 1, 0)]
    changed = jnp.logical_or(t == 0, eot_ref[t] != prev)

    @pl.when(changed)
    def _():
        w1s_ref[...] = w1_ref[0].astype(jnp.bfloat16)

    @pl.when(valid_ref[t] == 1)
    def _():
        h = jax.lax.dot_general(
            xs_ref[...].astype(jnp.bfloat16), w1s_ref[...],
            (((1,), (1,)), ((), ())),
            preferred_element_type=jnp.float32)
        h = h + b1_ref[0]
        h_ref[...] = (0.5 * h * (1.0 + jax.lax.erf(h * 0.7071067811865476))
                      ).astype(jnp.bfloat16)


def _ffn2_kernel(eot_ref, valid_ref, h_ref, w2_ref, b2_ref, ys_ref, w2s_ref):
    t = pl.program_id(0)
    prev = eot_ref[jnp.maximum(t - 1, 0)]
    changed = jnp.logical_or(t == 0, eot_ref[t] != prev)

    @pl.when(changed)
    def _():
        w2s_ref[...] = w2_ref[0].astype(jnp.bfloat16)

    @pl.when(valid_ref[t] == 1)
    def _():
        eo = jax.lax.dot_general(
            h_ref[...], w2s_ref[...], (((1,), (1,)), ((), ())),
            preferred_element_type=jnp.float32)
        ys_ref[...] = eo + b2_ref[0]


def _combine_kernel(y01_ref, w1c_ref, w2c_ref, out_ref):
    out_ref[...] = (w1c_ref[...] * y01_ref[:N] +
                    w2c_ref[...] * y01_ref[N:])


_vector_mesh = functools.partial(
    plsc.VectorSubcoreMesh, core_axis_name="c", subcore_axis_name="s")


def _sc_dispatch(xf, dest3):
    xv = xf.reshape(N * RSPLIT, DSUB)
    nwin = N * RSPLIT // SCW
    @pl.kernel(out_type=jax.ShapeDtypeStruct((PAD * RSPLIT, DSUB),
                                             jnp.float32),
               mesh=_vector_mesh())
    def disp(xv_hbm, d3_hbm, xs_hbm):
        def body(x_vmem, i_vmem):
            pltpu.sync_copy(x_vmem, xs_hbm.at[i_vmem.at[0]])

        pltpu.emit_pipeline(
            body,
            grid=(2 * nwin,),
            in_specs=[
                pl.BlockSpec((SCW, DSUB), lambda i: (i % nwin, 0)),
                pl.BlockSpec((1, SCW), lambda i: (0, i)),
            ],
            out_specs=[],
            core_axis_name=("c", "s"),
            dimension_semantics=(pltpu.PARALLEL,),
        )(xv_hbm, d3_hbm)

    return disp(xv, dest3).reshape(PAD, D)


def _sc_combine(ys, src3):
    yv = ys.reshape(PAD * RSPLIT, DSUB)
    nwin = NSLOT * N * RSPLIT // SCW
    @pl.kernel(out_type=jax.ShapeDtypeStruct((NSLOT * N * RSPLIT, DSUB),
                                             jnp.float32),
               mesh=_vector_mesh())
    def comb(yv_hbm, s3_hbm, y_hbm):
        def body(i_vmem, y_vmem):
            pltpu.sync_copy(yv_hbm.at[i_vmem.at[0]], y_vmem)

        pltpu.emit_pipeline(
            body,
            grid=(nwin,),
            in_specs=[
                pl.BlockSpec((1, SCW), lambda i: (0, i)),
            ],
            out_specs=[
                pl.BlockSpec((SCW, DSUB), lambda i: (i, 0)),
            ],
            core_axis_name=("c", "s"),
            dimension_semantics=(pltpu.PARALLEL,),
        )(s3_hbm, y_hbm)

    return comb(yv, src3).reshape(NSLOT * N, D)


_T128 = np.tril(np.ones((CH, CH), np.float32), -1)


def kernel(x, Wg, W1, b1, W2, b2):
    b, s, d = x.shape
    xf = x.reshape(N, D)

    t128 = jnp.asarray(_T128, jnp.bfloat16)
    w_full, w1c, w2c, posa, eot, valid = pl.pallas_call(
        _gate_route_kernel,
        out_shape=(
            jax.ShapeDtypeStruct((N, E), jnp.float32),
            jax.ShapeDtypeStruct((N, 1), jnp.float32),
            jax.ShapeDtypeStruct((N, 1), jnp.float32),
            jax.ShapeDtypeStruct((NCH, CH, RSPLIT), jnp.int32),
            jax.ShapeDtypeStruct((NT, 1), jnp.int32),
            jax.ShapeDtypeStruct((NT, 1), jnp.int32),
        ),
    )(xf, Wg.T, t128)

    idx3 = posa.reshape(1, A * RSPLIT)

    xs = _sc_dispatch(xf, idx3)

    b1r = b1.reshape(E, 1, H)
    b2r = b2.reshape(E, 1, D)
    eotf = eot.reshape(NT)
    validf = valid.reshape(NT)
    hs = pl.pallas_call(
        _ffn1_kernel,
        grid_spec=pltpu.PrefetchScalarGridSpec(
            num_scalar_prefetch=2,
            grid=(NT,),
            in_specs=[
                pl.BlockSpec((TM, D), lambda t, eot, valid: (t, 0)),
                pl.BlockSpec((1, H, D), lambda t, eot, valid: (eot[t], 0, 0)),
                pl.BlockSpec((1, 1, H), lambda t, eot, valid: (eot[t], 0, 0)),
            ],
            out_specs=pl.BlockSpec((TM, H), lambda t, eot, valid: (t, 0)),
            scratch_shapes=[pltpu.VMEM((H, D), jnp.bfloat16)],
        ),
        out_shape=jax.ShapeDtypeStruct((PAD, H), jnp.bfloat16),
    )(eotf, validf, xs, W1, b1r)

    ys = pl.pallas_call(
        _ffn2_kernel,
        grid_spec=pltpu.PrefetchScalarGridSpec(
            num_scalar_prefetch=2,
            grid=(NT,),
            in_specs=[
                pl.BlockSpec((TM, H), lambda t, eot, valid: (t, 0)),
                pl.BlockSpec((1, D, H), lambda t, eot, valid: (eot[t], 0, 0)),
                pl.BlockSpec((1, 1, D), lambda t, eot, valid: (eot[t], 0, 0)),
            ],
            out_specs=pl.BlockSpec((TM, D), lambda t, eot, valid: (t, 0)),
            scratch_shapes=[pltpu.VMEM((D, H), jnp.bfloat16)],
        ),
        out_shape=jax.ShapeDtypeStruct((PAD, D), jnp.float32),
    )(eotf, validf, hs, W2, b2r)

    y01 = _sc_combine(ys, idx3)

    out = pl.pallas_call(
        _combine_kernel,
        out_shape=jax.ShapeDtypeStruct((N, D), jnp.float32),
    )(y01, w1c, w2c)

    return out.reshape(b, s, d), w_full
